# Initial kernel scaffold; baseline (speedup 1.0000x reference)
import jax
import jax.numpy as jnp
from jax.experimental import pallas as pl


def kernel(bin_indices, embedding_weight):
    raise NotImplementedError("write your pallas kernel here")


# speedup vs baseline: 21.1967x; 21.1967x over previous
import functools

import jax
import jax.numpy as jnp
from jax import lax
from jax.experimental import pallas as pl
from jax.experimental.pallas import tpu as pltpu
from jax.experimental.pallas import tpu_sc as plsc

BATCH = 16384
BINS = 50
DIM = 16

NUM_CORES = 2
NUM_SUBCORES = 16
NUM_WORKERS = NUM_CORES * NUM_SUBCORES

ROWS_PER_WORKER = BATCH // NUM_WORKERS
CHUNK = 64
CHUNKS_PER_WORKER = ROWS_PER_WORKER // CHUNK
IDX_PER_CHUNK = CHUNK * BINS
IDX_COLS = 128
IDX_ROWS_PER_CHUNK = IDX_PER_CHUNK // IDX_COLS
IDX_ROWS_PER_WORKER = ROWS_PER_WORKER * BINS // IDX_COLS

_mesh = plsc.VectorSubcoreMesh(core_axis_name="c", subcore_axis_name="s")


@functools.partial(
    pl.kernel,
    mesh=_mesh,
    compiler_params=pltpu.CompilerParams(use_tc_tiling_on_sc=False),
    out_type=jax.ShapeDtypeStruct((BATCH, DIM), jnp.float32),
    scratch_types=[
        pltpu.VMEM((IDX_ROWS_PER_WORKER, IDX_COLS), jnp.int32),
        pltpu.VMEM((IDX_PER_CHUNK, DIM), jnp.float32),
        pltpu.VMEM((CHUNK, DIM), jnp.float32),
        pltpu.SemaphoreType.DMA,
    ],
)
def _pooled_lookup(table_hbm, idx_hbm, out_hbm, idx_v, rows_v, out_v, sem):
    wid = lax.axis_index("s") * NUM_CORES + lax.axis_index("c")

    pltpu.sync_copy(
        idx_hbm.at[pl.ds(wid * IDX_ROWS_PER_WORKER, IDX_ROWS_PER_WORKER)], idx_v
    )

    def chunk_body(g, carry):
        out_base = wid * ROWS_PER_WORKER + g * CHUNK
        idx_base = g * IDX_ROWS_PER_CHUNK

        copies = [
            pltpu.async_copy(
                table_hbm.at[idx_v.at[idx_base + j]],
                rows_v.at[pl.ds(j * IDX_COLS, IDX_COLS)],
                sem,
            )
            for j in range(IDX_ROWS_PER_CHUNK)
        ]
        for c in copies:
            c.wait()

        def acc_body(i, carry2):
            r = i * BINS
            acc = rows_v[r, :]
            for j in range(1, BINS):
                acc = acc + rows_v[r + j, :]
            out_v[i, :] = acc * jnp.float32(1.0 / BINS)
            return carry2

        lax.fori_loop(0, CHUNK, acc_body, 0)
        pltpu.sync_copy(out_v, out_hbm.at[pl.ds(out_base, CHUNK)])
        return carry

    lax.fori_loop(0, CHUNKS_PER_WORKER, chunk_body, 0)


def kernel(bin_indices, embedding_weight):
    idx2d = bin_indices.astype(jnp.int32).reshape(
        BATCH * BINS // IDX_COLS, IDX_COLS
    )
    return _pooled_lookup(embedding_weight, idx2d)


# speedup vs baseline: 28.1557x; 1.3283x over previous
import functools

import jax
import jax.numpy as jnp
from jax import lax
from jax.experimental import pallas as pl
from jax.experimental.pallas import tpu as pltpu
from jax.experimental.pallas import tpu_sc as plsc

BATCH = 16384
BINS = 50
DIM = 16

NUM_CORES = 2
NUM_SUBCORES = 16
NUM_WORKERS = NUM_CORES * NUM_SUBCORES

ROWS_PER_WORKER = BATCH // NUM_WORKERS
QUARTER = 128
NQ = ROWS_PER_WORKER // QUARTER

_mesh = plsc.VectorSubcoreMesh(core_axis_name="c", subcore_axis_name="s")


@functools.partial(
    pl.kernel,
    mesh=_mesh,
    compiler_params=pltpu.CompilerParams(use_tc_tiling_on_sc=False),
    out_type=jax.ShapeDtypeStruct((BATCH, DIM), jnp.float32),
    scratch_types=[
        pltpu.VMEM((BINS, ROWS_PER_WORKER), jnp.int32),
        pltpu.VMEM((ROWS_PER_WORKER, DIM), jnp.float32),
        pltpu.SemaphoreType.DMA,
    ],
)
def _pooled_lookup(table_hbm, idxt_hbm, out_hbm, idx_v, acc_v, sem):
    wid = lax.axis_index("s") * NUM_CORES + lax.axis_index("c")
    out_base = wid * ROWS_PER_WORKER

    pltpu.sync_copy(idxt_hbm.at[:, pl.ds(out_base, ROWS_PER_WORKER)], idx_v)

    def zero_body(i, carry):
        acc_v[i, :] = jnp.zeros((DIM,), jnp.float32)
        return carry

    lax.fori_loop(0, ROWS_PER_WORKER, zero_body, 0)

    copies = [
        pltpu.async_copy(
            table_hbm.at[idx_v.at[j, pl.ds(q * QUARTER, QUARTER)]],
            acc_v.at[pl.ds(q * QUARTER, QUARTER)],
            sem,
            add=True,
        )
        for q in range(NQ)
        for j in range(BINS)
    ]
    for c in copies:
        c.wait()

    def scale_body(i, carry):
        acc_v[i, :] = acc_v[i, :] * jnp.float32(1.0 / BINS)
        return carry

    lax.fori_loop(0, ROWS_PER_WORKER, scale_body, 0)
    pltpu.sync_copy(acc_v, out_hbm.at[pl.ds(out_base, ROWS_PER_WORKER)])


def kernel(bin_indices, embedding_weight):
    idxt = bin_indices.astype(jnp.int32).T
    return _pooled_lookup(embedding_weight, idxt)
